# baseline (speedup 1.0000x reference)
import jax
import jax.numpy as jnp
from jax import lax
from jax.experimental import pallas as pl
from jax.experimental.pallas import tpu as pltpu
from jax.experimental.pallas import tpu_sc as plsc

_NC = 2
_NS = 16
_NW = _NC * _NS
_CHUNK = 128
_ECHUNK = 64
_LANES = 16


def _sc_mesh():
    return plsc.VectorSubcoreMesh(core_axis_name="c", subcore_axis_name="s")


def _degree_kernel(npad, ept):
    def body(src_hbm, dst_hbm, out_hbm, idxs, idxd, hs, hd):
        c = lax.axis_index("c")
        s = lax.axis_index("s")
        wid = c * _NS + s

        pltpu.sync_copy(src_hbm.at[wid], idxs)
        pltpu.sync_copy(dst_hbm.at[wid], idxd)

        @pl.loop(0, npad, step=_LANES)
        def _(r):
            hs[pl.ds(r, _LANES)] = jnp.zeros((_LANES,), jnp.float32)
            hd[pl.ds(r, _LANES)] = jnp.zeros((_LANES,), jnp.float32)

        ones = jnp.ones((_LANES,), jnp.float32)

        @pl.loop(0, ept, step=_LANES)
        def _(k):
            plsc.addupdate_scatter(hs, [idxs[pl.ds(k, _LANES)]], ones)
            plsc.addupdate_scatter(hd, [idxd[pl.ds(k, _LANES)]], ones)

        pltpu.sync_copy(hs, out_hbm.at[0, wid])
        pltpu.sync_copy(hd, out_hbm.at[1, wid])

    return pl.kernel(
        body,
        out_type=jax.ShapeDtypeStruct((2, _NW, npad), jnp.float32),
        mesh=_sc_mesh(),
        compiler_params=pltpu.CompilerParams(needs_layout_passes=False),
        scratch_types=[
            pltpu.VMEM((ept,), jnp.int32),
            pltpu.VMEM((ept,), jnp.int32),
            pltpu.VMEM((npad,), jnp.float32),
            pltpu.VMEM((npad,), jnp.float32),
        ],
    )


def _norms_kernel(hists, blkn):
    npad = hists.shape[2]

    def body(h_ref, ns_ref, nd_ref):
        ds_ = jnp.sum(h_ref[0], axis=0)
        dd_ = jnp.sum(h_ref[1], axis=0)
        ns = lax.rsqrt(jnp.maximum(ds_, 1.0))[:, None]
        nd_ = lax.rsqrt(jnp.maximum(dd_, 1.0))[:, None]
        ns_ref[...] = jnp.broadcast_to(ns, (blkn, _LANES))
        nd_ref[...] = jnp.broadcast_to(nd_, (blkn, _LANES))

    return pl.pallas_call(
        body,
        grid=(npad // blkn,),
        in_specs=[pl.BlockSpec((2, _NW, blkn), lambda i: (0, 0, i))],
        out_specs=[
            pl.BlockSpec((blkn, _LANES), lambda i: (i, 0)),
            pl.BlockSpec((blkn, _LANES), lambda i: (i, 0)),
        ],
        out_shape=[
            jax.ShapeDtypeStruct((npad, _LANES), jnp.float32),
            jax.ShapeDtypeStruct((npad, _LANES), jnp.float32),
        ],
    )(hists)


def _aggregate_kernel(npad, d, k1, kx):
    nrows = npad // _NS
    nbuf = 4
    kmax = max(k1, kx)

    def body(table_hbm, sd_hbm, out_hbm, sdti, r0, r1, r2, r3,
             gs0, gs1, gs2, gs3, ss0, ss1, ss2, ss3, acc):
        c = lax.axis_index("c")
        s = lax.axis_index("s")
        wid = c * _NS + s
        rbase = s * nrows
        rows = (r0, r1, r2, r3)
        gsem = (gs0, gs1, gs2, gs3)
        ssem = (ss0, ss1, ss2, ss3)

        @pl.loop(0, _ECHUNK)
        def _(i):
            @pl.loop(0, d, step=_LANES)
            def _(j):
                r0[i, pl.ds(j, _LANES)] = jnp.zeros((_LANES,), jnp.float32)

        @pl.loop(0, nrows, step=_ECHUNK)
        def _(r):
            pltpu.sync_copy(r0, acc.at[pl.ds(rbase + r, _ECHUNK)])

        def start_g(b, j):
            pltpu.async_copy(table_hbm.at[sdti.at[j, pl.ds(0, _ECHUNK)]],
                             rows[b], gsem[b])

        def wait_g(b, j):
            pltpu.make_async_copy(table_hbm.at[sdti.at[j, pl.ds(0, _ECHUNK)]],
                                  rows[b], gsem[b]).wait()

        def start_s(b, j):
            pltpu.async_copy(rows[b], acc.at[sdti.at[j, pl.ds(_ECHUNK, _ECHUNK)]],
                             ssem[b], add=True)

        def wait_s(b, j):
            pltpu.make_async_copy(rows[b], acc.at[sdti.at[j, pl.ds(_ECHUNK, _ECHUNK)]],
                                  ssem[b]).wait()

        def pipeline(knch):
            for b in range(nbuf):
                start_g(b, b)

            @pl.loop(0, knch - nbuf, step=nbuf)
            def _(j0):
                for b in range(nbuf):
                    wait_g(b, j0 + b)
                    start_s(b, j0 + b)
                for b in range(nbuf):
                    wait_s(b, j0 + b)
                    start_g(b, j0 + nbuf + b)

            for b in range(nbuf):
                wait_g(b, knch - nbuf + b)
                start_s(b, knch - nbuf + b)
            for b in range(nbuf):
                wait_s(b, knch - nbuf + b)

        plsc.subcore_barrier()

        pltpu.sync_copy(sd_hbm.at[wid], sdti)
        pipeline(k1)

        @pl.when(c == 0)
        def _():
            pltpu.sync_copy(sd_hbm.at[_NW + s], sdti)
            pipeline(kx)

        plsc.subcore_barrier()

        @pl.loop(0, nrows, step=_CHUNK)
        def _(r):
            pltpu.sync_copy(acc.at[pl.ds(rbase + r, _CHUNK)],
                            out_hbm.at[c, pl.ds(rbase + r, _CHUNK)])

    return pl.kernel(
        body,
        out_type=jax.ShapeDtypeStruct((_NC, npad, d), jnp.float32),
        mesh=_sc_mesh(),
        scratch_types=[
            pltpu.VMEM((kmax, 2 * _ECHUNK), jnp.int32),
            pltpu.VMEM((_ECHUNK, d), jnp.float32),
            pltpu.VMEM((_ECHUNK, d), jnp.float32),
            pltpu.VMEM((_ECHUNK, d), jnp.float32),
            pltpu.VMEM((_ECHUNK, d), jnp.float32),
            pltpu.SemaphoreType.DMA,
            pltpu.SemaphoreType.DMA,
            pltpu.SemaphoreType.DMA,
            pltpu.SemaphoreType.DMA,
            pltpu.SemaphoreType.DMA,
            pltpu.SemaphoreType.DMA,
            pltpu.SemaphoreType.DMA,
            pltpu.SemaphoreType.DMA,
            pltpu.VMEM_SHARED((npad, d), jnp.float32),
        ],
    )


def _norm_cols(tab_ref):
    return tab_ref[...][:, 0:1]


def _mm2_kernel(x, w0, rw0, blk):
    n, d = x.shape

    def body(x_ref, w_ref, rw_ref, o1_ref, o2_ref):
        xb = x_ref[...]
        o1_ref[...] = jnp.dot(xb, w_ref[...], preferred_element_type=jnp.float32)
        o2_ref[...] = jnp.dot(xb, rw_ref[...], preferred_element_type=jnp.float32)

    return pl.pallas_call(
        body,
        grid=(n // blk,),
        in_specs=[
            pl.BlockSpec((blk, d), lambda i: (i, 0)),
            pl.BlockSpec((d, d), lambda i: (0, 0)),
            pl.BlockSpec((d, d), lambda i: (0, 0)),
        ],
        out_specs=[
            pl.BlockSpec((blk, d), lambda i: (i, 0)),
            pl.BlockSpec((blk, d), lambda i: (i, 0)),
        ],
        out_shape=[
            jax.ShapeDtypeStruct((n, d), jnp.float32),
            jax.ShapeDtypeStruct((n, d), jnp.float32),
        ],
    )(x, w0, rw0)


def _build0_kernel(xw0, xrw0, deg_s, rb0, blk):
    n, d = xw0.shape

    def body(xw_ref, xrw_ref, ds_ref, rb_ref, hs_ref, r_ref):
        ns = _norm_cols(ds_ref)
        hs_ref[...] = xw_ref[...] * ns
        r_ref[...] = xrw_ref[...] + rb_ref[...]

    return pl.pallas_call(
        body,
        grid=(n // blk,),
        in_specs=[
            pl.BlockSpec((blk, d), lambda i: (i, 0)),
            pl.BlockSpec((blk, d), lambda i: (i, 0)),
            pl.BlockSpec((blk, _LANES), lambda i: (i, 0)),
            pl.BlockSpec((1, d), lambda i: (0, 0)),
        ],
        out_specs=[
            pl.BlockSpec((blk, d), lambda i: (i, 0)),
            pl.BlockSpec((blk, d), lambda i: (i, 0)),
        ],
        out_shape=[
            jax.ShapeDtypeStruct((n, d), jnp.float32),
            jax.ShapeDtypeStruct((n, d), jnp.float32),
        ],
    )(xw0, xrw0, deg_s, rb0)


def _combine_kernel(agg, deg_s, deg_d, b, r_in, w_next, rw_next, rb_next, blk):
    n = r_in.shape[0]
    d = r_in.shape[1]

    def body(agg_ref, ds_ref, dd_ref, b_ref, r_ref, w_ref, rw_ref, rb_ref,
             hs_ref, rn_ref):
        nd_ = _norm_cols(dd_ref)
        a = agg_ref[0] + agg_ref[1]
        h = jnp.maximum(a * nd_ + b_ref[...], 0.0) + r_ref[...]
        ns = _norm_cols(ds_ref)
        hs_ref[...] = jnp.dot(h, w_ref[...],
                              preferred_element_type=jnp.float32) * ns
        rn_ref[...] = jnp.dot(h, rw_ref[...],
                              preferred_element_type=jnp.float32) + rb_ref[...]

    return pl.pallas_call(
        body,
        grid=(n // blk,),
        in_specs=[
            pl.BlockSpec((_NC, blk, d), lambda i: (0, i, 0)),
            pl.BlockSpec((blk, _LANES), lambda i: (i, 0)),
            pl.BlockSpec((blk, _LANES), lambda i: (i, 0)),
            pl.BlockSpec((1, d), lambda i: (0, 0)),
            pl.BlockSpec((blk, d), lambda i: (i, 0)),
            pl.BlockSpec((d, d), lambda i: (0, 0)),
            pl.BlockSpec((d, d), lambda i: (0, 0)),
            pl.BlockSpec((1, d), lambda i: (0, 0)),
        ],
        out_specs=[
            pl.BlockSpec((blk, d), lambda i: (i, 0)),
            pl.BlockSpec((blk, d), lambda i: (i, 0)),
        ],
        out_shape=[
            jax.ShapeDtypeStruct((n, d), jnp.float32),
            jax.ShapeDtypeStruct((n, d), jnp.float32),
        ],
    )(agg, deg_s, deg_d, b, r_in, w_next, rw_next, rb_next)


def _combine2_kernel(agg, deg_s, deg_d, b, r_in, w_next, blk):
    n = r_in.shape[0]
    d = r_in.shape[1]

    def body(agg_ref, ds_ref, dd_ref, b_ref, r_ref, w_ref, hs_ref, h_ref):
        nd_ = _norm_cols(dd_ref)
        a = agg_ref[0] + agg_ref[1]
        h = jnp.maximum(a * nd_ + b_ref[...], 0.0) + r_ref[...]
        ns = _norm_cols(ds_ref)
        h_ref[...] = h
        hs_ref[...] = jnp.dot(h, w_ref[...],
                              preferred_element_type=jnp.float32) * ns

    return pl.pallas_call(
        body,
        grid=(n // blk,),
        in_specs=[
            pl.BlockSpec((_NC, blk, d), lambda i: (0, i, 0)),
            pl.BlockSpec((blk, _LANES), lambda i: (i, 0)),
            pl.BlockSpec((blk, _LANES), lambda i: (i, 0)),
            pl.BlockSpec((1, d), lambda i: (0, 0)),
            pl.BlockSpec((blk, d), lambda i: (i, 0)),
            pl.BlockSpec((d, d), lambda i: (0, 0)),
        ],
        out_specs=[
            pl.BlockSpec((blk, d), lambda i: (i, 0)),
            pl.BlockSpec((blk, d), lambda i: (i, 0)),
        ],
        out_shape=[
            jax.ShapeDtypeStruct((n, d), jnp.float32),
            jax.ShapeDtypeStruct((n, d), jnp.float32),
        ],
    )(agg, deg_s, deg_d, b, r_in, w_next)


def _final_kernel(agg, deg_d, b, h2, blk):
    n, d = h2.shape

    def body(agg_ref, dd_ref, b_ref, h_ref, o_ref):
        nd_ = _norm_cols(dd_ref)
        a = agg_ref[0] + agg_ref[1]
        o_ref[...] = a * nd_ + b_ref[...] + h_ref[...]

    return pl.pallas_call(
        body,
        grid=(n // blk,),
        in_specs=[
            pl.BlockSpec((_NC, blk, d), lambda i: (0, i, 0)),
            pl.BlockSpec((blk, _LANES), lambda i: (i, 0)),
            pl.BlockSpec((1, d), lambda i: (0, 0)),
            pl.BlockSpec((blk, d), lambda i: (i, 0)),
        ],
        out_specs=pl.BlockSpec((blk, d), lambda i: (i, 0)),
        out_shape=jax.ShapeDtypeStruct((n, d), jnp.float32),
    )(agg, deg_d, b, h2)


def kernel(x, edge_index, W0, b0, W1, b1, W2, b2, RW0, Rb0, RW1, Rb1):
    n, d = x.shape
    e = edge_index.shape[1]

    npad = ((n + _CHUNK * _NS - 1) // (_CHUNK * _NS)) * (_CHUNK * _NS)
    ktot = ((e + _NS * _ECHUNK - 1) // (_NS * _ECHUNK) + 11) // 12 * 12
    k1 = ktot // 12 * 4
    kx = ktot - 2 * k1
    kmax = max(k1, kx)
    tot_chunks = ktot * _NS
    epad = tot_chunks * _ECHUNK
    ept = epad // _NW
    pad = epad - e

    src = edge_index[0]
    dst = edge_index[1]
    junk = n + (jnp.arange(pad, dtype=jnp.int32) % (npad - n))
    src_g = jnp.concatenate([src, jnp.zeros((pad,), jnp.int32)])
    src_d = jnp.concatenate([src, junk])
    dst_p = jnp.concatenate([dst, junk])
    sd_flat = jnp.concatenate([src_g.reshape(tot_chunks, _ECHUNK),
                               dst_p.reshape(tot_chunks, _ECHUNK)], axis=1)
    slack = jnp.concatenate(
        [jnp.zeros((_NW + _NS, kmax, _ECHUNK), jnp.int32),
         jnp.full((_NW + _NS, kmax, _ECHUNK), n, jnp.int32)], axis=2)
    sd3 = slack.at[:_NW, :k1].set(
        sd_flat[:_NW * k1].reshape(_NW, k1, 2 * _ECHUNK))
    sd3 = sd3.at[_NW:, :kx].set(
        sd_flat[_NW * k1:].reshape(_NS, kx, 2 * _ECHUNK))

    b0r = b0.reshape(1, d)
    b2r = b2.reshape(1, d)
    b1r = b1.reshape(1, d)
    rb0r = Rb0.reshape(1, d)
    rb1r = Rb1.reshape(1, d)

    blk = 1000 if n % 1000 == 0 else 8

    hists = _degree_kernel(npad, ept)(src_d.reshape(_NW, ept), dst_p.reshape(_NW, ept))
    xw0, xrw0 = _mm2_kernel(x, W0, RW0, blk)
    deg_s, deg_d = _norms_kernel(hists, 1024)

    agg_fn = _aggregate_kernel(npad, d, k1, kx)

    hs0, r0 = _build0_kernel(xw0, xrw0, deg_s, rb0r, blk)
    agg0 = agg_fn(hs0, sd3)
    hs1, r1 = _combine_kernel(agg0, deg_s, deg_d, b0r, r0, W1, RW1, rb1r, blk)
    agg1 = agg_fn(hs1, sd3)
    hs2, h2 = _combine2_kernel(agg1, deg_s, deg_d, b1r, r1, W2, blk)
    agg2 = agg_fn(hs2, sd3)
    return _final_kernel(agg2, deg_d, b2r, h2, blk)


# speedup vs baseline: 1.8138x; 1.8138x over previous
import jax
import jax.numpy as jnp
from jax import lax
from jax.experimental import pallas as pl
from jax.experimental.pallas import tpu as pltpu
from jax.experimental.pallas import tpu_sc as plsc

_NC = 2
_NS = 16
_NW = _NC * _NS
_CHUNK = 128
_ECHUNK = 64
_LANES = 16


def _sc_mesh():
    return plsc.VectorSubcoreMesh(core_axis_name="c", subcore_axis_name="s")


def _degree_kernel(npad, ept):
    def body(src_hbm, dst_hbm, out_hbm, idxs, idxd, hs, hd):
        c = lax.axis_index("c")
        s = lax.axis_index("s")
        wid = c * _NS + s

        pltpu.sync_copy(src_hbm.at[wid], idxs)
        pltpu.sync_copy(dst_hbm.at[wid], idxd)

        @pl.loop(0, npad, step=_LANES)
        def _(r):
            hs[pl.ds(r, _LANES)] = jnp.zeros((_LANES,), jnp.float32)
            hd[pl.ds(r, _LANES)] = jnp.zeros((_LANES,), jnp.float32)

        ones = jnp.ones((_LANES,), jnp.float32)

        @pl.loop(0, ept, step=_LANES)
        def _(k):
            plsc.addupdate_scatter(hs, [idxs[pl.ds(k, _LANES)]], ones)
            plsc.addupdate_scatter(hd, [idxd[pl.ds(k, _LANES)]], ones)

        pltpu.sync_copy(hs, out_hbm.at[0, wid])
        pltpu.sync_copy(hd, out_hbm.at[1, wid])

    return pl.kernel(
        body,
        out_type=jax.ShapeDtypeStruct((2, _NW, npad), jnp.float32),
        mesh=_sc_mesh(),
        compiler_params=pltpu.CompilerParams(needs_layout_passes=False),
        scratch_types=[
            pltpu.VMEM((ept,), jnp.int32),
            pltpu.VMEM((ept,), jnp.int32),
            pltpu.VMEM((npad,), jnp.float32),
            pltpu.VMEM((npad,), jnp.float32),
        ],
    )


def _norms_kernel(hists, blkn):
    npad = hists.shape[2]

    def body(h_ref, ns_ref, nd_ref):
        ds_ = jnp.sum(h_ref[0], axis=0)
        dd_ = jnp.sum(h_ref[1], axis=0)
        ns = lax.rsqrt(jnp.maximum(ds_, 1.0))[:, None]
        nd_ = lax.rsqrt(jnp.maximum(dd_, 1.0))[:, None]
        ns_ref[...] = jnp.broadcast_to(ns, (blkn, _LANES))
        nd_ref[...] = jnp.broadcast_to(nd_, (blkn, _LANES))

    return pl.pallas_call(
        body,
        grid=(npad // blkn,),
        in_specs=[pl.BlockSpec((2, _NW, blkn), lambda i: (0, 0, i))],
        out_specs=[
            pl.BlockSpec((blkn, _LANES), lambda i: (i, 0)),
            pl.BlockSpec((blkn, _LANES), lambda i: (i, 0)),
        ],
        out_shape=[
            jax.ShapeDtypeStruct((npad, _LANES), jnp.float32),
            jax.ShapeDtypeStruct((npad, _LANES), jnp.float32),
        ],
    )(hists)


def _aggregate_kernel(npad, d, k1, kx):
    nrows = npad // _NS
    nbuf = 3
    kmax = max(k1, kx)

    def body(table_hbm, sd_hbm, out_hbm, sdti, r0, r1, r2,
             gs0, gs1, gs2, ss0, ss1, ss2, acc):
        c = lax.axis_index("c")
        s = lax.axis_index("s")
        wid = c * _NS + s
        rbase = s * nrows
        rows = (r0, r1, r2)
        gsem = (gs0, gs1, gs2)
        ssem = (ss0, ss1, ss2)

        @pl.loop(0, _ECHUNK)
        def _(i):
            @pl.loop(0, d, step=_LANES)
            def _(j):
                r0[i, pl.ds(j, _LANES)] = jnp.zeros((_LANES,), jnp.float32)

        @pl.loop(0, nrows, step=_ECHUNK)
        def _(r):
            pltpu.sync_copy(r0, acc.at[pl.ds(rbase + r, _ECHUNK)])

        def start_g(b, j):
            pltpu.async_copy(table_hbm.at[sdti.at[j, pl.ds(0, _ECHUNK)]],
                             rows[b], gsem[b])

        def wait_g(b, j):
            pltpu.make_async_copy(table_hbm.at[sdti.at[j, pl.ds(0, _ECHUNK)]],
                                  rows[b], gsem[b]).wait()

        def start_s(b, j):
            pltpu.async_copy(rows[b], acc.at[sdti.at[j, pl.ds(_ECHUNK, _ECHUNK)]],
                             ssem[b], add=True)

        def wait_s(b, j):
            pltpu.make_async_copy(rows[b], acc.at[sdti.at[j, pl.ds(_ECHUNK, _ECHUNK)]],
                                  ssem[b]).wait()

        def pipeline(knch):
            for b in range(nbuf):
                start_g(b, b)

            @pl.loop(0, knch - nbuf, step=nbuf)
            def _(j0):
                for b in range(nbuf):
                    wait_g(b, j0 + b)
                    start_s(b, j0 + b)
                for b in range(nbuf):
                    wait_s(b, j0 + b)
                    start_g(b, j0 + nbuf + b)

            for b in range(nbuf):
                wait_g(b, knch - nbuf + b)
                start_s(b, knch - nbuf + b)
            for b in range(nbuf):
                wait_s(b, knch - nbuf + b)

        plsc.subcore_barrier()

        pltpu.sync_copy(sd_hbm.at[wid], sdti)
        pipeline(k1)

        if kx:
            @pl.when(c == 0)
            def _():
                pltpu.sync_copy(sd_hbm.at[_NW + s], sdti)
                pipeline(kx)

        plsc.subcore_barrier()

        @pl.loop(0, nrows, step=_CHUNK)
        def _(r):
            pltpu.sync_copy(acc.at[pl.ds(rbase + r, _CHUNK)],
                            out_hbm.at[c, pl.ds(rbase + r, _CHUNK)])

    return pl.kernel(
        body,
        out_type=jax.ShapeDtypeStruct((_NC, npad, d), jnp.float32),
        mesh=_sc_mesh(),
        scratch_types=[
            pltpu.VMEM((kmax, 2 * _ECHUNK), jnp.int32),
            pltpu.VMEM((_ECHUNK, d), jnp.float32),
            pltpu.VMEM((_ECHUNK, d), jnp.float32),
            pltpu.VMEM((_ECHUNK, d), jnp.float32),
            pltpu.SemaphoreType.DMA,
            pltpu.SemaphoreType.DMA,
            pltpu.SemaphoreType.DMA,
            pltpu.SemaphoreType.DMA,
            pltpu.SemaphoreType.DMA,
            pltpu.SemaphoreType.DMA,
            pltpu.VMEM_SHARED((npad, d), jnp.float32),
        ],
    )


def _norm_cols(tab_ref):
    return tab_ref[...][:, 0:1]


def _mm2_kernel(x, w0, rw0, blk):
    n, d = x.shape

    def body(x_ref, w_ref, rw_ref, o1_ref, o2_ref):
        xb = x_ref[...]
        o1_ref[...] = jnp.dot(xb, w_ref[...], preferred_element_type=jnp.float32)
        o2_ref[...] = jnp.dot(xb, rw_ref[...], preferred_element_type=jnp.float32)

    return pl.pallas_call(
        body,
        grid=(n // blk,),
        in_specs=[
            pl.BlockSpec((blk, d), lambda i: (i, 0)),
            pl.BlockSpec((d, d), lambda i: (0, 0)),
            pl.BlockSpec((d, d), lambda i: (0, 0)),
        ],
        out_specs=[
            pl.BlockSpec((blk, d), lambda i: (i, 0)),
            pl.BlockSpec((blk, d), lambda i: (i, 0)),
        ],
        out_shape=[
            jax.ShapeDtypeStruct((n, d), jnp.float32),
            jax.ShapeDtypeStruct((n, d), jnp.float32),
        ],
    )(x, w0, rw0)


def _build0_kernel(xw0, xrw0, deg_s, rb0, blk):
    n, d = xw0.shape

    def body(xw_ref, xrw_ref, ds_ref, rb_ref, hs_ref, r_ref):
        ns = _norm_cols(ds_ref)
        hs_ref[...] = xw_ref[...] * ns
        r_ref[...] = xrw_ref[...] + rb_ref[...]

    return pl.pallas_call(
        body,
        grid=(n // blk,),
        in_specs=[
            pl.BlockSpec((blk, d), lambda i: (i, 0)),
            pl.BlockSpec((blk, d), lambda i: (i, 0)),
            pl.BlockSpec((blk, _LANES), lambda i: (i, 0)),
            pl.BlockSpec((1, d), lambda i: (0, 0)),
        ],
        out_specs=[
            pl.BlockSpec((blk, d), lambda i: (i, 0)),
            pl.BlockSpec((blk, d), lambda i: (i, 0)),
        ],
        out_shape=[
            jax.ShapeDtypeStruct((n, d), jnp.float32),
            jax.ShapeDtypeStruct((n, d), jnp.float32),
        ],
    )(xw0, xrw0, deg_s, rb0)


def _combine_kernel(agg, deg_s, deg_d, b, r_in, w_next, rw_next, rb_next, blk):
    n = r_in.shape[0]
    d = r_in.shape[1]

    def body(agg_ref, ds_ref, dd_ref, b_ref, r_ref, w_ref, rw_ref, rb_ref,
             hs_ref, rn_ref):
        nd_ = _norm_cols(dd_ref)
        a = agg_ref[0] + agg_ref[1]
        h = jnp.maximum(a * nd_ + b_ref[...], 0.0) + r_ref[...]
        ns = _norm_cols(ds_ref)
        hs_ref[...] = jnp.dot(h, w_ref[...],
                              preferred_element_type=jnp.float32) * ns
        rn_ref[...] = jnp.dot(h, rw_ref[...],
                              preferred_element_type=jnp.float32) + rb_ref[...]

    return pl.pallas_call(
        body,
        grid=(n // blk,),
        in_specs=[
            pl.BlockSpec((_NC, blk, d), lambda i: (0, i, 0)),
            pl.BlockSpec((blk, _LANES), lambda i: (i, 0)),
            pl.BlockSpec((blk, _LANES), lambda i: (i, 0)),
            pl.BlockSpec((1, d), lambda i: (0, 0)),
            pl.BlockSpec((blk, d), lambda i: (i, 0)),
            pl.BlockSpec((d, d), lambda i: (0, 0)),
            pl.BlockSpec((d, d), lambda i: (0, 0)),
            pl.BlockSpec((1, d), lambda i: (0, 0)),
        ],
        out_specs=[
            pl.BlockSpec((blk, d), lambda i: (i, 0)),
            pl.BlockSpec((blk, d), lambda i: (i, 0)),
        ],
        out_shape=[
            jax.ShapeDtypeStruct((n, d), jnp.float32),
            jax.ShapeDtypeStruct((n, d), jnp.float32),
        ],
    )(agg, deg_s, deg_d, b, r_in, w_next, rw_next, rb_next)


def _combine2_kernel(agg, deg_s, deg_d, b, r_in, w_next, blk):
    n = r_in.shape[0]
    d = r_in.shape[1]

    def body(agg_ref, ds_ref, dd_ref, b_ref, r_ref, w_ref, hs_ref, h_ref):
        nd_ = _norm_cols(dd_ref)
        a = agg_ref[0] + agg_ref[1]
        h = jnp.maximum(a * nd_ + b_ref[...], 0.0) + r_ref[...]
        ns = _norm_cols(ds_ref)
        h_ref[...] = h
        hs_ref[...] = jnp.dot(h, w_ref[...],
                              preferred_element_type=jnp.float32) * ns

    return pl.pallas_call(
        body,
        grid=(n // blk,),
        in_specs=[
            pl.BlockSpec((_NC, blk, d), lambda i: (0, i, 0)),
            pl.BlockSpec((blk, _LANES), lambda i: (i, 0)),
            pl.BlockSpec((blk, _LANES), lambda i: (i, 0)),
            pl.BlockSpec((1, d), lambda i: (0, 0)),
            pl.BlockSpec((blk, d), lambda i: (i, 0)),
            pl.BlockSpec((d, d), lambda i: (0, 0)),
        ],
        out_specs=[
            pl.BlockSpec((blk, d), lambda i: (i, 0)),
            pl.BlockSpec((blk, d), lambda i: (i, 0)),
        ],
        out_shape=[
            jax.ShapeDtypeStruct((n, d), jnp.float32),
            jax.ShapeDtypeStruct((n, d), jnp.float32),
        ],
    )(agg, deg_s, deg_d, b, r_in, w_next)


def _final_kernel(agg, deg_d, b, h2, blk):
    n, d = h2.shape

    def body(agg_ref, dd_ref, b_ref, h_ref, o_ref):
        nd_ = _norm_cols(dd_ref)
        a = agg_ref[0] + agg_ref[1]
        o_ref[...] = a * nd_ + b_ref[...] + h_ref[...]

    return pl.pallas_call(
        body,
        grid=(n // blk,),
        in_specs=[
            pl.BlockSpec((_NC, blk, d), lambda i: (0, i, 0)),
            pl.BlockSpec((blk, _LANES), lambda i: (i, 0)),
            pl.BlockSpec((1, d), lambda i: (0, 0)),
            pl.BlockSpec((blk, d), lambda i: (i, 0)),
        ],
        out_specs=pl.BlockSpec((blk, d), lambda i: (i, 0)),
        out_shape=jax.ShapeDtypeStruct((n, d), jnp.float32),
    )(agg, deg_d, b, h2)


def kernel(x, edge_index, W0, b0, W1, b1, W2, b2, RW0, Rb0, RW1, Rb1):
    n, d = x.shape
    e = edge_index.shape[1]

    npad = ((n + _CHUNK * _NS - 1) // (_CHUNK * _NS)) * (_CHUNK * _NS)
    ktot = ((e + _NS * _ECHUNK - 1) // (_NS * _ECHUNK) + 5) // 6 * 6
    k1 = ktot // 2
    kx = 0
    kmax = max(k1, kx)
    tot_chunks = ktot * _NS
    epad = tot_chunks * _ECHUNK
    ept = epad // _NW
    pad = epad - e

    src = edge_index[0]
    dst = edge_index[1]
    junk = n + (jnp.arange(pad, dtype=jnp.int32) % (npad - n))
    src_g = jnp.concatenate([src, jnp.zeros((pad,), jnp.int32)])
    src_d = jnp.concatenate([src, junk])
    dst_p = jnp.concatenate([dst, junk])
    sd_flat = jnp.concatenate([src_g.reshape(tot_chunks, _ECHUNK),
                               dst_p.reshape(tot_chunks, _ECHUNK)], axis=1)
    slack = jnp.concatenate(
        [jnp.zeros((_NW + _NS, kmax, _ECHUNK), jnp.int32),
         jnp.full((_NW + _NS, kmax, _ECHUNK), n, jnp.int32)], axis=2)
    sd3 = slack.at[:_NW, :k1].set(
        sd_flat[:_NW * k1].reshape(_NW, k1, 2 * _ECHUNK))
    if kx:
        sd3 = sd3.at[_NW:, :kx].set(
            sd_flat[_NW * k1:].reshape(_NS, kx, 2 * _ECHUNK))

    b0r = b0.reshape(1, d)
    b2r = b2.reshape(1, d)
    b1r = b1.reshape(1, d)
    rb0r = Rb0.reshape(1, d)
    rb1r = Rb1.reshape(1, d)

    blk = 1000 if n % 1000 == 0 else 8

    hists = _degree_kernel(npad, ept)(src_d.reshape(_NW, ept), dst_p.reshape(_NW, ept))
    xw0, xrw0 = _mm2_kernel(x, W0, RW0, blk)
    deg_s, deg_d = _norms_kernel(hists, 1024)

    agg_fn = _aggregate_kernel(npad, d, k1, kx)

    hs0, r0 = _build0_kernel(xw0, xrw0, deg_s, rb0r, blk)
    agg0 = agg_fn(hs0, sd3)
    hs1, r1 = _combine_kernel(agg0, deg_s, deg_d, b0r, r0, W1, RW1, rb1r, blk)
    agg1 = agg_fn(hs1, sd3)
    hs2, h2 = _combine2_kernel(agg1, deg_s, deg_d, b1r, r1, W2, blk)
    agg2 = agg_fn(hs2, sd3)
    return _final_kernel(agg2, deg_d, b2r, h2, blk)
